# baseline (speedup 1.0000x reference)
import functools

import jax
import jax.numpy as jnp
from jax import lax
from jax.experimental import pallas as pl
from jax.experimental.pallas import tpu as pltpu
from jax.experimental.pallas import tpu_sc as plsc

_NC = 2
_NS = 16
_NW = _NC * _NS
_L = 16


def kernel(xs, input_table, gt_table):
    B = xs.shape[0]
    V, D = input_table.shape
    assert B % (8 * _NW) == 0 and D == _L
    b_per_w = B // _NW
    n_chunks = b_per_w // _L

    in_t = input_table.T
    gt_t = gt_table.T

    mesh = plsc.VectorSubcoreMesh(core_axis_name="c", subcore_axis_name="s")

    @functools.partial(
        pl.kernel,
        mesh=mesh,
        compiler_params=pltpu.CompilerParams(use_tc_tiling_on_sc=False),
        out_type=(
            jax.ShapeDtypeStruct((D, B), jnp.float32),
            jax.ShapeDtypeStruct((D, B), jnp.float32),
        ),
        scratch_types=[
            pltpu.VMEM((b_per_w,), jnp.float32),
            pltpu.VMEM((b_per_w,), jnp.int32),
            pltpu.VMEM((D, b_per_w), jnp.float32),
            pltpu.VMEM((D, b_per_w), jnp.float32),
            pltpu.SemaphoreType.DMA,
            pltpu.SemaphoreType.DMA,
        ],
    )
    def sc_kernel(xs_hbm, in_hbm, gt_hbm, out_in_hbm, out_gt_hbm,
                  xs_v, idx_v, outa_v, outb_v, sem_a, sem_b):
        wid = lax.axis_index("s") * _NC + lax.axis_index("c")
        base = wid * b_per_w

        pltpu.sync_copy(xs_hbm.at[pl.ds(base, b_per_w)], xs_v)

        scale = jnp.float32(V)
        hi = jnp.float32(V - 1)

        def idx_body(i, carry):
            v = xs_v[pl.ds(i * _L, _L)]
            scaled = v * scale
            clipped = jnp.minimum(jnp.maximum(scaled, jnp.float32(0.0)), hi)
            idx_v[pl.ds(i * _L, _L)] = clipped.astype(jnp.int32)
            return carry

        lax.fori_loop(0, n_chunks, idx_body, 0)

        copies = []
        for c in range(D):
            copies.append(pltpu.async_copy(
                in_hbm.at[c].at[idx_v], outa_v.at[c], sem_a))
            copies.append(pltpu.async_copy(
                gt_hbm.at[c].at[idx_v], outb_v.at[c], sem_b))
        for cp in copies:
            cp.wait()

        pltpu.sync_copy(outa_v, out_in_hbm.at[:, pl.ds(base, b_per_w)])
        pltpu.sync_copy(outb_v, out_gt_hbm.at[:, pl.ds(base, b_per_w)])

    out_in_t, out_gt_t = sc_kernel(xs, in_t, gt_t)
    return out_in_t.T, out_gt_t.T


# speedup vs baseline: 18.6001x; 18.6001x over previous
import functools

import jax
import jax.numpy as jnp
from jax import lax
from jax.experimental import pallas as pl
from jax.experimental.pallas import tpu as pltpu
from jax.experimental.pallas import tpu_sc as plsc

_NC = 2
_NS = 16
_NW = _NC * _NS
_L = 16
_WV = 8


def kernel(xs, input_table, gt_table):
    B = xs.shape[0]
    V, D = input_table.shape
    assert B % (2 * _WV * _NW) == 0 and D == _L
    b_per_w = B // _NW
    n_chunks = b_per_w // _L
    n_dwaves = b_per_w // (2 * _WV)
    max_blk = (V - 128) // 128 * 128
    tail_lo = max_blk + 128
    n_tail = V - tail_lo

    in_t = input_table.T
    gt_t = gt_table.T
    tail_in = input_table[tail_lo:].reshape(-1)
    tail_gt = gt_table[tail_lo:].reshape(-1)

    mesh = plsc.VectorSubcoreMesh(core_axis_name="c", subcore_axis_name="s")

    @functools.partial(
        pl.kernel,
        mesh=mesh,
        compiler_params=pltpu.CompilerParams(needs_layout_passes=False),
        out_type=(
            jax.ShapeDtypeStruct((D, B), jnp.float32),
            jax.ShapeDtypeStruct((D, B), jnp.float32),
        ),
        scratch_types=[
            pltpu.VMEM((b_per_w,), jnp.float32),
            pltpu.VMEM((b_per_w,), jnp.int32),
            pltpu.VMEM((2 * _WV, D, 128), jnp.float32),
            pltpu.VMEM((2 * _WV, D, 128), jnp.float32),
            pltpu.VMEM((n_tail * D,), jnp.float32),
            pltpu.VMEM((n_tail * D,), jnp.float32),
            pltpu.VMEM((D * b_per_w,), jnp.float32),
            pltpu.VMEM((D * b_per_w,), jnp.float32),
            pltpu.SemaphoreType.DMA,
            pltpu.SemaphoreType.DMA,
            pltpu.SemaphoreType.DMA,
            pltpu.SemaphoreType.DMA,
        ],
    )
    def sc_kernel(xs_hbm, in_hbm, gt_hbm, tin_hbm, tgt_hbm,
                  out_in_hbm, out_gt_hbm,
                  xs_v, idx_v, bufa, bufb, taila_v, tailb_v,
                  outa_v, outb_v, sa0, sa1, sb0, sb1):
        wid = lax.axis_index("s") * _NC + lax.axis_index("c")
        base = wid * b_per_w

        pltpu.sync_copy(xs_hbm.at[pl.ds(base, b_per_w)], xs_v)
        pltpu.sync_copy(tin_hbm, taila_v)
        pltpu.sync_copy(tgt_hbm, tailb_v)

        scale = jnp.float32(V)
        hi = jnp.float32(V - 1)

        def idx_body(i, carry):
            v = xs_v[pl.ds(i * _L, _L)]
            scaled = v * scale
            clipped = jnp.minimum(jnp.maximum(scaled, jnp.float32(0.0)), hi)
            idx_v[pl.ds(i * _L, _L)] = clipped.astype(jnp.int32)
            return carry

        lax.fori_loop(0, n_chunks, idx_body, 0)

        iota16 = lax.iota(jnp.int32, _L)

        def lane_scalars(iv, lane):
            r_s = jnp.max(jnp.where(iota16 == lane, iv, 0))
            blk_s = jnp.minimum(r_s & jnp.int32(~127), jnp.int32(max_blk))
            return blk_s, r_s - blk_s

        def fire(iv, lane0, sem_a, sem_b, slot0):
            for s in range(_WV):
                blk_s, _ = lane_scalars(iv, lane0 + s)
                blk = pl.multiple_of(blk_s, 128)
                pltpu.async_copy(
                    in_hbm.at[:, pl.ds(blk, 128)], bufa.at[slot0 + s], sem_a)
                pltpu.async_copy(
                    gt_hbm.at[:, pl.ds(blk, 128)], bufb.at[slot0 + s], sem_b)

        def process(iv, i, lane0, sem_a, sem_b, slot0):
            for s in range(_WV):
                pltpu.make_async_copy(
                    in_hbm.at[:, pl.ds(0, 128)], bufa.at[slot0 + s],
                    sem_a).wait()
                pltpu.make_async_copy(
                    gt_hbm.at[:, pl.ds(0, 128)], bufb.at[slot0 + s],
                    sem_b).wait()
            zeros16 = jnp.zeros((_L,), jnp.int32)
            for s in range(_WV):
                lane = lane0 + s
                _, col_s = lane_scalars(iv, lane)
                cvec = zeros16 + col_s
                cclamp = jnp.minimum(cvec, jnp.int32(127))
                trow_s = jnp.maximum(col_s - 128, jnp.int32(0))
                tpos = zeros16 + trow_s * D + iota16
                svec = jnp.full((_L,), slot0 + s, jnp.int32)
                is_tail = cvec >= 128
                opos = iota16 * b_per_w + (i * _L + lane)
                va = plsc.load_gather(bufa, [svec, iota16, cclamp])
                vat = plsc.load_gather(taila_v, [tpos])
                plsc.store_scatter(outa_v, [opos], jnp.where(is_tail, vat, va))
                vb = plsc.load_gather(bufb, [svec, iota16, cclamp])
                vbt = plsc.load_gather(tailb_v, [tpos])
                plsc.store_scatter(outb_v, [opos], jnp.where(is_tail, vbt, vb))

        def chunk_iv(i):
            return idx_v[pl.ds(i * _L, _L)]

        def dwave(i, carry):
            iv = chunk_iv(i)
            fire(iv, 0, sa0, sb0, 0)

            @pl.when(i >= 1)
            def _():
                process(chunk_iv(i - 1), i - 1, _WV, sa1, sb1, _WV)

            fire(iv, _WV, sa1, sb1, _WV)
            process(iv, i, 0, sa0, sb0, 0)
            return carry

        lax.fori_loop(0, n_dwaves, dwave, 0)
        process(chunk_iv(n_dwaves - 1), n_dwaves - 1, _WV, sa1, sb1, _WV)

        for c in range(D):
            pltpu.sync_copy(outa_v.at[pl.ds(c * b_per_w, b_per_w)],
                            out_in_hbm.at[c, pl.ds(base, b_per_w)])
            pltpu.sync_copy(outb_v.at[pl.ds(c * b_per_w, b_per_w)],
                            out_gt_hbm.at[c, pl.ds(base, b_per_w)])

    out_in_t, out_gt_t = sc_kernel(xs, in_t, gt_t, tail_in, tail_gt)
    return out_in_t.T, out_gt_t.T


# speedup vs baseline: 18.6296x; 1.0016x over previous
import functools

import jax
import jax.numpy as jnp
from jax import lax
from jax.experimental import pallas as pl
from jax.experimental.pallas import tpu as pltpu
from jax.experimental.pallas import tpu_sc as plsc

_NC = 2
_NS = 16
_NW = _NC * _NS
_L = 16
_WV = 8


def kernel(xs, input_table, gt_table):
    B = xs.shape[0]
    V, D = input_table.shape
    assert B % (2 * _WV * _NW) == 0 and D == _L
    b_per_w = B // _NW
    n_chunks = b_per_w // _L
    n_dwaves = b_per_w // (2 * _WV)
    max_blk = (V - 128) // 128 * 128
    tail_lo = max_blk + 128
    n_tail = V - tail_lo

    in_t = input_table.T
    gt_t = gt_table.T
    tail_in = input_table[tail_lo:].reshape(-1)
    tail_gt = gt_table[tail_lo:].reshape(-1)

    mesh = plsc.VectorSubcoreMesh(core_axis_name="c", subcore_axis_name="s")

    @functools.partial(
        pl.kernel,
        mesh=mesh,
        compiler_params=pltpu.CompilerParams(needs_layout_passes=False),
        out_type=(
            jax.ShapeDtypeStruct((D, B), jnp.float32),
            jax.ShapeDtypeStruct((D, B), jnp.float32),
        ),
        scratch_types=[
            pltpu.VMEM((b_per_w,), jnp.float32),
            pltpu.VMEM((b_per_w,), jnp.int32),
            pltpu.VMEM((2 * _WV, D, 128), jnp.float32),
            pltpu.VMEM((2 * _WV, D, 128), jnp.float32),
            pltpu.VMEM((n_tail * D,), jnp.float32),
            pltpu.VMEM((n_tail * D,), jnp.float32),
            pltpu.VMEM((D * b_per_w,), jnp.float32),
            pltpu.VMEM((D * b_per_w,), jnp.float32),
            pltpu.SemaphoreType.DMA,
            pltpu.SemaphoreType.DMA,
            pltpu.SemaphoreType.DMA,
            pltpu.SemaphoreType.DMA,
        ],
    )
    def sc_kernel(xs_hbm, in_hbm, gt_hbm, tin_hbm, tgt_hbm,
                  out_in_hbm, out_gt_hbm,
                  xs_v, idx_v, bufa, bufb, taila_v, tailb_v,
                  outa_v, outb_v, sa0, sa1, sb0, sb1):
        wid = lax.axis_index("s") * _NC + lax.axis_index("c")
        base = wid * b_per_w

        pltpu.sync_copy(xs_hbm.at[pl.ds(base, b_per_w)], xs_v)
        pltpu.sync_copy(tin_hbm, taila_v)
        pltpu.sync_copy(tgt_hbm, tailb_v)

        scale = jnp.float32(V)
        hi = jnp.float32(V - 1)

        def idx_body(i, carry):
            v = xs_v[pl.ds(i * _L, _L)]
            scaled = v * scale
            clipped = jnp.minimum(jnp.maximum(scaled, jnp.float32(0.0)), hi)
            idx_v[pl.ds(i * _L, _L)] = clipped.astype(jnp.int32)
            return carry

        lax.fori_loop(0, n_chunks, idx_body, 0)

        iota16 = lax.iota(jnp.int32, _L)

        def lane_scalars(iv, lane):
            r_s = jnp.max(jnp.where(iota16 == lane, iv, 0))
            blk_s = jnp.minimum(r_s & jnp.int32(~127), jnp.int32(max_blk))
            return blk_s, r_s - blk_s

        def fire(scal, sem_a, sem_b, slot0):
            for s in range(_WV):
                blk = pl.multiple_of(scal[s][0], 128)
                pltpu.async_copy(
                    in_hbm.at[:, pl.ds(blk, 128)], bufa.at[slot0 + s], sem_a)
                pltpu.async_copy(
                    gt_hbm.at[:, pl.ds(blk, 128)], bufb.at[slot0 + s], sem_b)

        def process(scal, has_tail, i, lane0, sem_a, sem_b, slot0):
            drain_a = pltpu.make_async_copy(
                in_hbm.at[:, pl.ds(0, 128)], bufa.at[slot0], sem_a)
            drain_b = pltpu.make_async_copy(
                gt_hbm.at[:, pl.ds(0, 128)], bufb.at[slot0], sem_b)
            for s in range(_WV):
                drain_a.wait()
                drain_b.wait()
            zeros16 = jnp.zeros((_L,), jnp.int32)
            for s in range(_WV):
                lane = lane0 + s
                col_s = scal[s][1]
                cvec = zeros16 + col_s
                cclamp = jnp.minimum(cvec, jnp.int32(127))
                svec = jnp.full((_L,), slot0 + s, jnp.int32)
                opos = iota16 * b_per_w + (i * _L + lane)
                va = plsc.load_gather(bufa, [svec, iota16, cclamp])
                plsc.store_scatter(outa_v, [opos], va)
                vb = plsc.load_gather(bufb, [svec, iota16, cclamp])
                plsc.store_scatter(outb_v, [opos], vb)

            @pl.when(has_tail)
            def _():
                for s in range(_WV):
                    lane = lane0 + s
                    col_s = scal[s][1]
                    cvec = zeros16 + col_s
                    is_tail = cvec >= 128
                    trow_s = jnp.maximum(col_s - 128, jnp.int32(0))
                    tpos = zeros16 + trow_s * D + iota16
                    opos = iota16 * b_per_w + (i * _L + lane)
                    vat = plsc.load_gather(taila_v, [tpos])
                    plsc.store_scatter(outa_v, [opos], vat, mask=is_tail)
                    vbt = plsc.load_gather(tailb_v, [tpos])
                    plsc.store_scatter(outb_v, [opos], vbt, mask=is_tail)

        def chunk_scal(i):
            iv = idx_v[pl.ds(i * _L, _L)]
            scal = [lane_scalars(iv, l) for l in range(_L)]
            has_tail = jnp.max(iv) >= jnp.int32(tail_lo)
            return scal, has_tail

        def dwave(i, carry):
            scal, has_tail = chunk_scal(i)
            fire(scal[:_WV], sa0, sb0, 0)

            @pl.when(i >= 1)
            def _():
                scp, htp = chunk_scal(i - 1)
                process(scp[_WV:], htp, i - 1, _WV, sa1, sb1, _WV)

            fire(scal[_WV:], sa1, sb1, _WV)
            process(scal[:_WV], has_tail, i, 0, sa0, sb0, 0)
            return carry

        lax.fori_loop(0, n_dwaves, dwave, 0)
        scl, htl = chunk_scal(n_dwaves - 1)
        process(scl[_WV:], htl, n_dwaves - 1, _WV, sa1, sb1, _WV)

        for c in range(D):
            pltpu.sync_copy(outa_v.at[pl.ds(c * b_per_w, b_per_w)],
                            out_in_hbm.at[c, pl.ds(base, b_per_w)])
            pltpu.sync_copy(outb_v.at[pl.ds(c * b_per_w, b_per_w)],
                            out_gt_hbm.at[c, pl.ds(base, b_per_w)])

    out_in_t, out_gt_t = sc_kernel(xs, in_t, gt_t, tail_in, tail_gt)
    return out_in_t.T, out_gt_t.T


# speedup vs baseline: 20.1058x; 1.0792x over previous
import functools

import jax
import jax.numpy as jnp
from jax import lax
from jax.experimental import pallas as pl
from jax.experimental.pallas import tpu as pltpu
from jax.experimental.pallas import tpu_sc as plsc

_NC = 2
_NS = 16
_NW = _NC * _NS
_L = 16
_WV = 8


def kernel(xs, input_table, gt_table):
    B = xs.shape[0]
    V, D = input_table.shape
    assert B % (2 * _WV * _NW) == 0 and D == _L
    b_per_w = B // _NW
    n_chunks = b_per_w // _L
    n_dwaves = b_per_w // (2 * _WV)
    max_blk = (V - 128) // 128 * 128
    tail_lo = max_blk + 128
    n_tail = V - tail_lo

    in_t = input_table.T
    gt_t = gt_table.T
    tail_in = input_table[tail_lo:].reshape(-1)
    tail_gt = gt_table[tail_lo:].reshape(-1)

    mesh = plsc.VectorSubcoreMesh(core_axis_name="c", subcore_axis_name="s")

    @functools.partial(
        pl.kernel,
        mesh=mesh,
        compiler_params=pltpu.CompilerParams(needs_layout_passes=False),
        out_type=(
            jax.ShapeDtypeStruct((D, B), jnp.float32),
            jax.ShapeDtypeStruct((D, B), jnp.float32),
        ),
        scratch_types=[
            pltpu.VMEM((b_per_w,), jnp.float32),
            pltpu.VMEM((b_per_w,), jnp.int32),
            pltpu.VMEM((3 * _WV, D, 128), jnp.float32),
            pltpu.VMEM((3 * _WV, D, 128), jnp.float32),
            pltpu.VMEM((n_tail * D,), jnp.float32),
            pltpu.VMEM((n_tail * D,), jnp.float32),
            pltpu.VMEM((D * b_per_w,), jnp.float32),
            pltpu.VMEM((D * b_per_w,), jnp.float32),
            pltpu.SemaphoreType.DMA,
            pltpu.SemaphoreType.DMA,
            pltpu.SemaphoreType.DMA,
            pltpu.SemaphoreType.DMA,
            pltpu.SemaphoreType.DMA,
            pltpu.SemaphoreType.DMA,
        ],
    )
    def sc_kernel(xs_hbm, in_hbm, gt_hbm, tin_hbm, tgt_hbm,
                  out_in_hbm, out_gt_hbm,
                  xs_v, idx_v, bufa, bufb, taila_v, tailb_v,
                  outa_v, outb_v, sa0, sa1, sa2, sb0, sb1, sb2):
        wid = lax.axis_index("s") * _NC + lax.axis_index("c")
        base = wid * b_per_w

        pltpu.sync_copy(xs_hbm.at[pl.ds(base, b_per_w)], xs_v)
        pltpu.sync_copy(tin_hbm, taila_v)
        pltpu.sync_copy(tgt_hbm, tailb_v)

        scale = jnp.float32(V)
        hi = jnp.float32(V - 1)

        def idx_body(i, carry):
            v = xs_v[pl.ds(i * _L, _L)]
            scaled = v * scale
            clipped = jnp.minimum(jnp.maximum(scaled, jnp.float32(0.0)), hi)
            idx_v[pl.ds(i * _L, _L)] = clipped.astype(jnp.int32)
            return carry

        lax.fori_loop(0, n_chunks, idx_body, 0)

        iota16 = lax.iota(jnp.int32, _L)

        def lane_scalars(iv, lane):
            r_s = jnp.max(jnp.where(iota16 == lane, iv, 0))
            blk_s = jnp.minimum(r_s & jnp.int32(~127), jnp.int32(max_blk))
            return blk_s, r_s - blk_s

        def fire(scal, sem_a, sem_b, slot0):
            for s in range(_WV):
                blk = pl.multiple_of(scal[s][0], 128)
                pltpu.async_copy(
                    in_hbm.at[:, pl.ds(blk, 128)], bufa.at[slot0 + s], sem_a)
                pltpu.async_copy(
                    gt_hbm.at[:, pl.ds(blk, 128)], bufb.at[slot0 + s], sem_b)

        def process(scal, has_tail, i, lane0, sem_a, sem_b, slot0):
            drain_a = pltpu.make_async_copy(
                in_hbm.at[:, pl.ds(0, 128)], bufa.at[slot0], sem_a)
            drain_b = pltpu.make_async_copy(
                gt_hbm.at[:, pl.ds(0, 128)], bufb.at[slot0], sem_b)
            for s in range(_WV):
                drain_a.wait()
                drain_b.wait()
            zeros16 = jnp.zeros((_L,), jnp.int32)
            for s in range(_WV):
                lane = lane0 + s
                col_s = scal[s][1]
                cvec = zeros16 + col_s
                cclamp = jnp.minimum(cvec, jnp.int32(127))
                svec = jnp.full((_L,), slot0 + s, jnp.int32)
                opos = iota16 * b_per_w + (i * _L + lane)
                va = plsc.load_gather(bufa, [svec, iota16, cclamp])
                plsc.store_scatter(outa_v, [opos], va)
                vb = plsc.load_gather(bufb, [svec, iota16, cclamp])
                plsc.store_scatter(outb_v, [opos], vb)

            @pl.when(has_tail)
            def _():
                for s in range(_WV):
                    lane = lane0 + s
                    col_s = scal[s][1]
                    cvec = zeros16 + col_s
                    is_tail = cvec >= 128
                    trow_s = jnp.maximum(col_s - 128, jnp.int32(0))
                    tpos = zeros16 + trow_s * D + iota16
                    opos = iota16 * b_per_w + (i * _L + lane)
                    vat = plsc.load_gather(taila_v, [tpos])
                    plsc.store_scatter(outa_v, [opos], vat, mask=is_tail)
                    vbt = plsc.load_gather(tailb_v, [tpos])
                    plsc.store_scatter(outb_v, [opos], vbt, mask=is_tail)

        sems_a = (sa0, sa1, sa2)
        sems_b = (sb0, sb1, sb2)
        n_waves = b_per_w // _WV

        def wave_scal(w):
            iv = idx_v[pl.ds((w >> 1) * _L, _L)]
            lane0 = (w & 1) * _WV
            scal = [lane_scalars(iv, lane0 + s) for s in range(_WV)]
            has_tail = jnp.max(iv) >= jnp.int32(tail_lo)
            return scal, has_tail, lane0

        def fire_w(w, p):
            scal, _, _ = wave_scal(w)
            fire(scal, sems_a[p], sems_b[p], p * _WV)

        def process_w(w, p):
            scal, has_tail, lane0 = wave_scal(w)
            process(scal, has_tail, w >> 1, lane0,
                    sems_a[p], sems_b[p], p * _WV)

        fire_w(0, 0)
        fire_w(1, 1)

        def trip(i, carry):
            for p in range(3):
                w = 3 * i + p + 2

                @pl.when(w <= n_waves - 1)
                def _(w=w, p=p):
                    fire_w(w, (p + 2) % 3)

                process_w(w - 2, p)
            return carry

        lax.fori_loop(0, (n_waves + 1) // 3, trip, 0)
        process_w(n_waves - 1, (n_waves - 1) % 3)

        for c in range(D):
            pltpu.sync_copy(outa_v.at[pl.ds(c * b_per_w, b_per_w)],
                            out_in_hbm.at[c, pl.ds(base, b_per_w)])
            pltpu.sync_copy(outb_v.at[pl.ds(c * b_per_w, b_per_w)],
                            out_gt_hbm.at[c, pl.ds(base, b_per_w)])

    out_in_t, out_gt_t = sc_kernel(xs, in_t, gt_t, tail_in, tail_gt)
    return out_in_t.T, out_gt_t.T
